# baseline (speedup 1.0000x reference)
import jax
import jax.numpy as jnp
from jax import lax
from jax.experimental import pallas as pl
from jax.experimental.pallas import tpu as pltpu
from jax.experimental.pallas import tpu_sc as plsc

NUM_FIELDS = 26
VOCAB = 100000
EMBED_DIM = 32
BATCH = 1024
TIME = 50

NW = 32
NPLANES = NUM_FIELDS * EMBED_DIM
PL_PER_W = NPLANES // NW
T_CHUNKS = [(0, 8), (8, 8), (16, 8), (24, 8), (32, 8), (40, 8), (48, 2)]
G_OF_CHUNK = [((0, 4), (4, 4))] * 6 + [((0, 2),)]


def _body(x_hbm, tab_hbm, out_hbm,
          vocab_v, xc0, xc1, oc0, oc1, vsem, vsem2, xsem, osem):
    wid = lax.axis_index("s") * 2 + lax.axis_index("c")
    xc = [xc0, xc1]
    oc = [oc0, oc1]

    def plane_fd(j):
        p = wid * PL_PER_W + j
        return p // EMBED_DIM, p % EMBED_DIM

    def fire_vocab(f, d):
        pltpu.async_copy(tab_hbm.at[f, d, :], vocab_v, vsem)

    def fire_x(f, t0, tc, cb):
        pltpu.async_copy(x_hbm.at[f, pl.ds(t0, tc), :],
                         xc[cb].at[pl.ds(0, tc)], xsem)

    def wait_x(tc, cb):
        pltpu.make_async_copy(x_hbm.at[0, pl.ds(0, tc), :],
                              xc[cb].at[pl.ds(0, tc)], xsem).wait()

    def drain_store(ob, gsz):
        pltpu.make_async_copy(
            oc[ob].at[pl.ds(0, gsz)],
            out_hbm.at[pl.ds(0, gsz), 0, 0, :], osem).wait()

    def plane_body(j, _):
        f, d = plane_fd(j)
        pltpu.make_async_copy(tab_hbm.at[0, 0, :], vocab_v, vsem).wait()

        si = 0
        for ci, (t0, tc) in enumerate(T_CHUNKS):
            cb = ci % 2
            wait_x(tc, cb)
            if ci + 1 < len(T_CHUNKS):
                nt0, ntc = T_CHUNKS[ci + 1]
                fire_x(f, nt0, ntc, 1 - cb)

            for g0, gsz in G_OF_CHUNK[ci]:
                ob = si % 2
                if si == 0:
                    @pl.when(j > 0)
                    def _():
                        drain_store(0, 2)
                elif si == 1:
                    @pl.when(j > 0)
                    def _():
                        drain_store(1, 4)
                else:
                    drain_store(ob, 4)

                def gather_t(t, _, cb=cb, ob=ob, g0=g0):
                    for k in range(BATCH // 16):
                        idxv = xc[cb][g0 + t, pl.ds(k * 16, 16)]
                        oc[ob][t, pl.ds(k * 16, 16)] = plsc.load_gather(
                            vocab_v, [idxv])
                    return 0

                lax.fori_loop(0, gsz, gather_t, 0, unroll=2)
                pltpu.async_copy(
                    oc[ob].at[pl.ds(0, gsz)],
                    out_hbm.at[pl.ds(t0 + g0, gsz), f, d, :], osem)
                si += 1

        @pl.when(j + 1 < PL_PER_W)
        def _():
            nf, nd = plane_fd(j + 1)
            fire_vocab(nf, nd)
            fire_x(nf, 0, 8, 0)
        return 0

    f0, d0 = plane_fd(0)
    fire_vocab(f0, d0)
    fire_x(f0, 0, 8, 0)

    lax.fori_loop(0, PL_PER_W, plane_body, 0, unroll=False)

    drain_store(1, 4)
    drain_store(0, 2)


@jax.jit
def kernel(x, tables):
    x_t = x.transpose(2, 1, 0)
    tab_t = tables.transpose(0, 2, 1)

    mesh = plsc.VectorSubcoreMesh(core_axis_name="c", subcore_axis_name="s")
    out = pl.kernel(
        _body,
        out_type=jax.ShapeDtypeStruct((TIME, NUM_FIELDS, EMBED_DIM, BATCH),
                                      jnp.float32),
        mesh=mesh,
        compiler_params=pltpu.CompilerParams(needs_layout_passes=False,
                                             use_tc_tiling_on_sc=True),
        scratch_types=[
            pltpu.VMEM((VOCAB,), jnp.float32),
            pltpu.VMEM((8, BATCH), jnp.int32),
            pltpu.VMEM((8, BATCH), jnp.int32),
            pltpu.VMEM((4, BATCH), jnp.float32),
            pltpu.VMEM((4, BATCH), jnp.float32),
            pltpu.SemaphoreType.DMA,
            pltpu.SemaphoreType.DMA,
            pltpu.SemaphoreType.DMA,
            pltpu.SemaphoreType.DMA,
        ],
    )(x_t, tab_t)
    return out.transpose(3, 0, 2, 1)


# speedup vs baseline: 1.0104x; 1.0104x over previous
import jax
import jax.numpy as jnp
from jax import lax
from jax.experimental import pallas as pl
from jax.experimental.pallas import tpu as pltpu
from jax.experimental.pallas import tpu_sc as plsc

NUM_FIELDS = 26
VOCAB = 100000
EMBED_DIM = 32
BATCH = 1024
TIME = 50

NW = 32
NPLANES = NUM_FIELDS * EMBED_DIM
PL_PER_W = NPLANES // NW
T_CHUNKS = [(0, 8), (8, 8), (16, 8), (24, 8), (32, 8), (40, 8), (48, 2)]
G_OF_CHUNK = [((0, 4), (4, 4))] * 6 + [((0, 2),)]


def _body(x_hbm, tab_hbm, out_hbm,
          vocab_v, xc0, xc1, oc0, oc1, vsem, xsem, osem):
    wid = lax.axis_index("s") * 2 + lax.axis_index("c")
    xc = [xc0, xc1]
    oc = [oc0, oc1]

    def plane_fd(j):
        p = wid * PL_PER_W + j
        return p // EMBED_DIM, p % EMBED_DIM

    def fire_vocab(f, d):
        pltpu.async_copy(tab_hbm.at[f, d, :], vocab_v, vsem)

    def fire_x(f, t0, tc, cb):
        pltpu.async_copy(x_hbm.at[f, pl.ds(t0, tc), :],
                         xc[cb].at[pl.ds(0, tc)], xsem)

    def wait_x(tc, cb):
        pltpu.make_async_copy(x_hbm.at[0, pl.ds(0, tc), :],
                              xc[cb].at[pl.ds(0, tc)], xsem).wait()

    def drain_store(ob, gsz):
        pltpu.make_async_copy(
            oc[ob].at[pl.ds(0, gsz)],
            out_hbm.at[pl.ds(0, gsz), 0, 0, :], osem).wait()

    def plane_body(j, _):
        f, d = plane_fd(j)
        pltpu.make_async_copy(tab_hbm.at[0, 0, :], vocab_v, vsem).wait()

        si = 0
        for ci, (t0, tc) in enumerate(T_CHUNKS):
            cb = ci % 2
            wait_x(tc, cb)
            if ci + 1 < len(T_CHUNKS):
                nt0, ntc = T_CHUNKS[ci + 1]
                fire_x(f, nt0, ntc, 1 - cb)

            for g0, gsz in G_OF_CHUNK[ci]:
                ob = si % 2
                if si == 0:
                    @pl.when(j > 0)
                    def _():
                        drain_store(0, 2)
                elif si == 1:
                    @pl.when(j > 0)
                    def _():
                        drain_store(1, 4)
                else:
                    drain_store(ob, 4)

                def gather_t(t, _, cb=cb, ob=ob, g0=g0):
                    for k in range(BATCH // 16):
                        idxv = xc[cb][g0 + t, pl.ds(k * 16, 16)]
                        oc[ob][t, pl.ds(k * 16, 16)] = plsc.load_gather(
                            vocab_v, [idxv])
                    return 0

                lax.fori_loop(0, gsz, gather_t, 0, unroll=False)
                pltpu.async_copy(
                    oc[ob].at[pl.ds(0, gsz)],
                    out_hbm.at[pl.ds(t0 + g0, gsz), f, d, :], osem)
                si += 1

        @pl.when(j + 1 < PL_PER_W)
        def _():
            nf, nd = plane_fd(j + 1)
            fire_vocab(nf, nd)
            fire_x(nf, 0, 8, 0)
        return 0

    f0, d0 = plane_fd(0)
    fire_vocab(f0, d0)
    fire_x(f0, 0, 8, 0)

    lax.fori_loop(0, PL_PER_W, plane_body, 0, unroll=False)

    drain_store(1, 4)
    drain_store(0, 2)


@jax.jit
def kernel(x, tables):
    x_t = x.transpose(2, 1, 0)
    tab_t = tables.transpose(0, 2, 1)

    mesh = plsc.VectorSubcoreMesh(core_axis_name="c", subcore_axis_name="s")
    out = pl.kernel(
        _body,
        out_type=jax.ShapeDtypeStruct((TIME, NUM_FIELDS, EMBED_DIM, BATCH),
                                      jnp.float32),
        mesh=mesh,
        compiler_params=pltpu.CompilerParams(needs_layout_passes=False,
                                             use_tc_tiling_on_sc=True),
        scratch_types=[
            pltpu.VMEM((VOCAB,), jnp.float32),
            pltpu.VMEM((8, BATCH), jnp.int32),
            pltpu.VMEM((8, BATCH), jnp.int32),
            pltpu.VMEM((4, BATCH), jnp.float32),
            pltpu.VMEM((4, BATCH), jnp.float32),
            pltpu.SemaphoreType.DMA,
            pltpu.SemaphoreType.DMA,
            pltpu.SemaphoreType.DMA,
        ],
    )(x_t, tab_t)
    return out.transpose(3, 0, 2, 1)
